# baseline (speedup 1.0000x reference)
import functools

import jax
import jax.numpy as jnp
from jax import lax
from jax.experimental import pallas as pl
from jax.experimental.pallas import tpu as pltpu
from jax.experimental.pallas import tpu_sc as plsc

_USER_NUM = 1000000
_NC = 2
_NS = 16
_NW = _NC * _NS
_L = 16
_E = 16
_CHUNK = 128


def _ffm_body(user_hbm, item_hbm, fcT_hbm, bias_hbm, emb0T_hbm, emb1T_hbm,
              out_hbm, u_idx, i_idx, gu, gi, fc_u, fc_i, bias_v, out_v,
              sem_u, sem_i, sem_g, b_per_w):
    wid = lax.axis_index("s") * _NC + lax.axis_index("c")
    base = wid * b_per_w
    n_chunks = b_per_w // _CHUNK

    cp_u = pltpu.async_copy(user_hbm.at[pl.ds(base, b_per_w)], u_idx, sem_u)
    cp_i = pltpu.async_copy(item_hbm.at[pl.ds(base, b_per_w)], i_idx, sem_i)
    pltpu.sync_copy(bias_hbm, bias_v)

    gathers = []
    cp_u.wait()
    for c in range(n_chunks):
        sl = pl.ds(c * _CHUNK, _CHUNK)
        usl = u_idx.at[sl]
        gathers.append(pltpu.async_copy(fcT_hbm.at[0].at[usl],
                                        fc_u.at[sl], sem_g))
        for k in range(_E):
            gathers.append(pltpu.async_copy(emb1T_hbm.at[k].at[usl],
                                            gu.at[k].at[sl], sem_g))
    cp_i.wait()
    for v in range(b_per_w // _L):
        sl = pl.ds(v * _L, _L)
        i_idx[sl] = i_idx[sl] + _USER_NUM
    for c in range(n_chunks):
        sl = pl.ds(c * _CHUNK, _CHUNK)
        isl = i_idx.at[sl]
        gathers.append(pltpu.async_copy(fcT_hbm.at[0].at[isl],
                                        fc_i.at[sl], sem_g))
        for k in range(_E):
            gathers.append(pltpu.async_copy(emb0T_hbm.at[k].at[isl],
                                            gi.at[k].at[sl], sem_g))
    for g in gathers:
        g.wait()

    bias_bc = bias_v[...]
    for v in range(b_per_w // _L):
        sl = pl.ds(v * _L, _L)
        acc = fc_u[sl] + fc_i[sl] + bias_bc
        for k in range(_E):
            acc = acc + gu[k, sl] * gi[k, sl]
        out_v[sl] = acc

    pltpu.sync_copy(out_v, out_hbm.at[pl.ds(base, b_per_w)])


def kernel(user, item, features, fc, bias, emb0, emb1):
    del features
    b = user.shape[0]
    b_per_w = b // _NW
    mesh = plsc.VectorSubcoreMesh(core_axis_name="c", subcore_axis_name="s")
    run = pl.kernel(
        functools.partial(_ffm_body, b_per_w=b_per_w),
        out_type=jax.ShapeDtypeStruct((b,), jnp.float32),
        mesh=mesh,
        scratch_types=[
            pltpu.VMEM((b_per_w,), jnp.int32),
            pltpu.VMEM((b_per_w,), jnp.int32),
            pltpu.VMEM((_E, b_per_w), jnp.float32),
            pltpu.VMEM((_E, b_per_w), jnp.float32),
            pltpu.VMEM((b_per_w,), jnp.float32),
            pltpu.VMEM((b_per_w,), jnp.float32),
            pltpu.VMEM((_L,), jnp.float32),
            pltpu.VMEM((b_per_w,), jnp.float32),
            pltpu.SemaphoreType.DMA,
            pltpu.SemaphoreType.DMA,
            pltpu.SemaphoreType.DMA,
        ],
        compiler_params=pltpu.CompilerParams(
            needs_layout_passes=False, use_tc_tiling_on_sc=False),
    )
    bias16 = jnp.broadcast_to(bias, (_L,))
    return run(user, item, fc.T, bias16, emb0.T, emb1.T)


# speedup vs baseline: 3.5322x; 3.5322x over previous
import functools

import jax
import jax.numpy as jnp
from jax import lax
from jax.experimental import pallas as pl
from jax.experimental.pallas import tpu as pltpu
from jax.experimental.pallas import tpu_sc as plsc

_USER_NUM = 1000000
_NC = 2
_NS = 16
_NW = _NC * _NS
_L = 16
_CHUNK = 128


def _ffm_body(user_hbm, item_hbm, fc_hbm, bias_hbm, emb0_hbm, emb1_hbm,
              out_hbm, u_idx, i_idx, rows_u, rows_i, fc_u, fc_i, bias_v,
              out_v, sem_u, sem_i, sem_g, b_per_w):
    wid = lax.axis_index("s") * _NC + lax.axis_index("c")
    base = wid * b_per_w
    n_chunks = b_per_w // _CHUNK
    n_groups = b_per_w // _L

    cp_u = pltpu.async_copy(user_hbm.at[pl.ds(base, b_per_w)], u_idx, sem_u)
    cp_i = pltpu.async_copy(item_hbm.at[pl.ds(base, b_per_w)], i_idx, sem_i)
    pltpu.sync_copy(bias_hbm, bias_v)

    cp_u.wait()
    gathers = []
    for j in range(n_chunks):
        sl = pl.ds(j * _CHUNK, _CHUNK)
        gathers.append(pltpu.async_copy(emb1_hbm.at[u_idx.at[sl]],
                                        rows_u.at[sl], sem_g))
        gathers.append(pltpu.async_copy(fc_hbm.at[u_idx.at[sl]],
                                        fc_u.at[sl], sem_g))

    cp_i.wait()
    for v in range(b_per_w // _L):
        sl = pl.ds(v * _L, _L)
        i_idx[sl] = i_idx[sl] + _USER_NUM
    for j in range(n_chunks):
        sl = pl.ds(j * _CHUNK, _CHUNK)
        gathers.append(pltpu.async_copy(emb0_hbm.at[i_idx.at[sl]],
                                        rows_i.at[sl], sem_g))
        gathers.append(pltpu.async_copy(fc_hbm.at[i_idx.at[sl]],
                                        fc_i.at[sl], sem_g))
    for g in gathers:
        g.wait()

    iota = lax.iota(jnp.int32, _L)
    bias_bc = bias_v[...]

    def group(g, _):
        rid = g * _L + iota
        acc = plsc.load_gather(fc_u, [rid]) + plsc.load_gather(fc_i, [rid])
        acc = acc + bias_bc
        for k in range(_L):
            kv = jnp.full((_L,), k, jnp.int32)
            a = plsc.load_gather(rows_u, [rid, kv])
            b = plsc.load_gather(rows_i, [rid, kv])
            acc = acc + a * b
        out_v[pl.ds(g * _L, _L)] = acc
        return _

    lax.fori_loop(0, n_groups, group, None)
    pltpu.sync_copy(out_v, out_hbm.at[pl.ds(base, b_per_w)])


def kernel(user, item, features, fc, bias, emb0, emb1):
    del features
    b = user.shape[0]
    b_per_w = b // _NW
    mesh = plsc.VectorSubcoreMesh(core_axis_name="c", subcore_axis_name="s")
    run = pl.kernel(
        functools.partial(_ffm_body, b_per_w=b_per_w),
        out_type=jax.ShapeDtypeStruct((b,), jnp.float32),
        mesh=mesh,
        scratch_types=[
            pltpu.VMEM((b_per_w,), jnp.int32),
            pltpu.VMEM((b_per_w,), jnp.int32),
            pltpu.VMEM((b_per_w, _L), jnp.float32),
            pltpu.VMEM((b_per_w, _L), jnp.float32),
            pltpu.VMEM((b_per_w,), jnp.float32),
            pltpu.VMEM((b_per_w,), jnp.float32),
            pltpu.VMEM((_L,), jnp.float32),
            pltpu.VMEM((b_per_w,), jnp.float32),
            pltpu.SemaphoreType.DMA,
            pltpu.SemaphoreType.DMA,
            pltpu.SemaphoreType.DMA,
        ],
        compiler_params=pltpu.CompilerParams(
            needs_layout_passes=False, use_tc_tiling_on_sc=False),
    )
    bias16 = jnp.broadcast_to(bias, (_L,))
    return run(user, item, fc.reshape(-1), bias16, emb0, emb1)


# speedup vs baseline: 7.3847x; 2.0907x over previous
import functools

import jax
import jax.numpy as jnp
from jax import lax
from jax.experimental import pallas as pl
from jax.experimental.pallas import tpu as pltpu
from jax.experimental.pallas import tpu_sc as plsc

_USER_NUM = 1000000
_V = 1100000
_NC = 2
_NS = 16
_L = 16
_KH = 8
_W = 4096
_PR = (_V + _L - 1) // _L
_CHUNK = 128
_UWIN0, _UWINS = 0, 245
_IWIN0, _IWINS = 244, 24
_IWINE = _IWIN0 + _IWINS
_ALIGNED_END = 1099904
_NTAIL = _V - _ALIGNED_END


def _repack_window(buf, col0, stage, dst, row0, n_rows, iota):
    kidx = jnp.bitwise_and(iota, 7)
    vsub = lax.shift_right_logical(iota, 3)

    def row(r, _):
        for sub in range(_KH):
            val = plsc.load_gather(
                buf, [kidx, col0 + r * _L + sub * 2 + vsub])
            plsc.store_scatter(
                stage, [jnp.full((_L,), 0, jnp.int32) + r,
                        sub * _L + iota], val)
        return _

    lax.fori_loop(0, n_rows, row, None)
    pltpu.sync_copy(stage.at[pl.ds(0, n_rows), :],
                    dst.at[pl.ds(row0, n_rows), :])


def _relay(src3, kh, dst, s, win0, n_win, wb, stage, sems, iota):
    n_t = (n_win + _NS - 1) // _NS
    end = win0 + n_win

    def win_id(t):
        return win0 + t * _NS + s

    def src(g):
        return src3.at[kh, :, pl.ds(g * _W, _W)]

    def process(t, par):
        @pl.when(win_id(t) < end)
        def _():
            g = win_id(t)
            pltpu.make_async_copy(src(g), wb[par], sems[par]).wait()
            for h in range(2):
                _repack_window(wb[par], h * 128 * _L, stage, dst,
                               g * (_W // _L) + h * 128, 128, iota)

    def fire(t, par):
        @pl.when(win_id(t) < end)
        def _():
            pltpu.async_copy(src(win_id(t)), wb[par], sems[par])

    fire(0, 0)

    def pair(tp, _):
        t = tp * 2
        fire(t + 1, 1)
        process(t, 0)
        fire(t + 2, 0)
        process(t + 1, 1)
        return _

    lax.fori_loop(0, (n_t + 1) // 2, pair, None)


def _item_tail(src3, kh, dst, s, wb, stage, iota):
    @pl.when(s == _NS - 1)
    def _():
        pltpu.sync_copy(src3.at[kh, :, pl.ds(_IWINE * _W, 2048)],
                        wb[0].at[:, pl.ds(0, 2048)])
        _repack_window(wb[0], 0, stage, dst, _IWINE * (_W // _L), 128, iota)

    @pl.when(s == _NS - 2)
    def _():
        off = _IWINE * _W + 2048
        pltpu.sync_copy(src3.at[kh, :, pl.ds(off, 128)],
                        wb[1].at[:, pl.ds(0, 128)])
        _repack_window(wb[1], 0, stage, dst, off // _L, 8, iota)


def _gather_compute(p_u, p_i, khof, on0, u_idx, i_idx, rv_u, rv_i, cb_u,
                    cb_i, rows_u, rows_i, fc_u, fc_i, bias_v, tail_v, out_v,
                    sem_u, sem_i, iota, b_per_w):
    def prep(v, _):
        sl = pl.ds(v * _L, _L)
        u = u_idx[sl]
        it = i_idx[sl]
        rv_u[sl] = lax.shift_right_logical(u, 4)
        cb_u[sl] = lax.shift_left(jnp.bitwise_and(u, 15), 3)
        rv_i[sl] = lax.shift_right_logical(it, 4)
        cb_i[sl] = lax.shift_left(jnp.bitwise_and(it, 15), 3)
        return _

    lax.fori_loop(0, b_per_w // _L, prep, None)

    bias_bc = bias_v[...]
    zeros = jnp.zeros((_L,), jnp.float32)
    def chunk(ch, _):
        sl = pl.ds(ch * _CHUNK, _CHUNK)
        gu = pltpu.async_copy(p_u.at[rv_u.at[sl]], rows_u, sem_u)
        gi = pltpu.async_copy(p_i.at[rv_i.at[sl]], rows_i, sem_i)
        gu.wait()
        gi.wait()

        def group(g, _):
            gsl = pl.ds(ch * _CHUNK + g * _L, _L)
            rid = g * _L + iota
            ub = cb_u[gsl]
            ib = cb_i[gsl]
            it = i_idx[gsl]
            in_tail = it >= _ALIGNED_END
            dv = khof + jnp.maximum(it - _ALIGNED_END, 0)
            acc = jnp.where(on0, fc_u[gsl] + fc_i[gsl] + bias_bc, zeros)
            for k in range(_KH):
                a = plsc.load_gather(rows_u, [rid, ub + k])
                b = plsc.load_gather(rows_i, [rid, ib + k])
                tb = plsc.load_gather(tail_v, [dv + k * _NTAIL])
                acc = acc + a * jnp.where(in_tail, tb, b)
            out_v[gsl] = acc
            return _

        lax.fori_loop(0, _CHUNK // _L, group, None)
        return _

    lax.fori_loop(0, b_per_w // _CHUNK, chunk, None)


def _ffm_body(user_hbm, item_hbm, fc_hbm, bias_hbm, tail_hbm, emb0_hbm,
              emb1_hbm, pu0, pi0, pu1, pi1, o0, o1, u_idx, i_idx, rv_u,
              rv_i, cb_u, cb_i, wb0, wb1, stage, rows_u, rows_i, fc_u,
              fc_i, bias_v, tail_v, out_v, sem_u, sem_i, sem_f, b_per_w):
    c = lax.axis_index("c")
    s = lax.axis_index("s")
    base = s * b_per_w
    iota = lax.iota(jnp.int32, _L)
    wb = (wb0, wb1)

    cp_u = pltpu.async_copy(user_hbm.at[pl.ds(base, b_per_w)], u_idx, sem_u)
    cp_i = pltpu.async_copy(item_hbm.at[pl.ds(base, b_per_w)], i_idx, sem_i)
    pltpu.sync_copy(bias_hbm, bias_v)
    cp_u.wait()
    cp_i.wait()

    def fix(v, _):
        sl = pl.ds(v * _L, _L)
        i_idx[sl] = i_idx[sl] + _USER_NUM
        return _

    lax.fori_loop(0, b_per_w // _L, fix, None)

    @pl.when(c == 0)
    def _():
        for ch in range(b_per_w // _CHUNK):
            sl = pl.ds(ch * _CHUNK, _CHUNK)
            pltpu.async_copy(fc_hbm.at[u_idx.at[sl]], fc_u.at[sl], sem_f)
            pltpu.async_copy(fc_hbm.at[i_idx.at[sl]], fc_i.at[sl], sem_f)

    pltpu.sync_copy(tail_hbm, tail_v)
    sems = (sem_u, sem_i)

    @pl.when(c == 0)
    def _():
        _relay(emb1_hbm, 0, pu0, s, _UWIN0, _UWINS, wb, stage, sems, iota)
        _relay(emb0_hbm, 0, pi0, s, _IWIN0, _IWINS, wb, stage, sems, iota)
        _item_tail(emb0_hbm, 0, pi0, s, wb, stage, iota)

    @pl.when(c == 1)
    def _():
        _relay(emb1_hbm, 1, pu1, s, _UWIN0, _UWINS, wb, stage, sems, iota)
        _relay(emb0_hbm, 1, pi1, s, _IWIN0, _IWINS, wb, stage, sems, iota)
        _item_tail(emb0_hbm, 1, pi1, s, wb, stage, iota)

    plsc.subcore_barrier()

    @pl.when(c == 0)
    def _():
        pltpu.make_async_copy(
            fc_hbm.at[pl.ds(0, b_per_w)], fc_u, sem_f).wait()
        pltpu.make_async_copy(
            fc_hbm.at[pl.ds(0, b_per_w)], fc_i, sem_f).wait()

    @pl.when(c == 0)
    def _():
        _gather_compute(pu0, pi0, 0, c == 0, u_idx, i_idx, rv_u, rv_i,
                        cb_u, cb_i, rows_u, rows_i, fc_u, fc_i, bias_v,
                        tail_v, out_v, sem_u, sem_i, iota, b_per_w)
        pltpu.sync_copy(out_v, o0.at[pl.ds(base, b_per_w)])

    @pl.when(c == 1)
    def _():
        _gather_compute(pu1, pi1, _KH * _NTAIL, c == 0, u_idx, i_idx,
                        rv_u, rv_i, cb_u, cb_i, rows_u, rows_i, fc_u,
                        fc_i, bias_v, tail_v, out_v, sem_u, sem_i, iota,
                        b_per_w)
        pltpu.sync_copy(out_v, o1.at[pl.ds(base, b_per_w)])


def kernel(user, item, features, fc, bias, emb0, emb1):
    del features
    b = user.shape[0]
    b_per_w = b // _NS
    mesh = plsc.VectorSubcoreMesh(core_axis_name="c", subcore_axis_name="s")
    pshape = jax.ShapeDtypeStruct((_PR, _KH * _L), jnp.float32)
    run = pl.kernel(
        functools.partial(_ffm_body, b_per_w=b_per_w),
        out_type=(pshape, pshape, pshape, pshape,
                  jax.ShapeDtypeStruct((b,), jnp.float32),
                  jax.ShapeDtypeStruct((b,), jnp.float32)),
        mesh=mesh,
        scratch_types=[
            pltpu.VMEM((b_per_w,), jnp.int32),
            pltpu.VMEM((b_per_w,), jnp.int32),
            pltpu.VMEM((b_per_w,), jnp.int32),
            pltpu.VMEM((b_per_w,), jnp.int32),
            pltpu.VMEM((b_per_w,), jnp.int32),
            pltpu.VMEM((b_per_w,), jnp.int32),
            pltpu.VMEM((_KH, _W), jnp.float32),
            pltpu.VMEM((_KH, _W), jnp.float32),
            pltpu.VMEM((128, _KH * _L), jnp.float32),
            pltpu.VMEM((_CHUNK, _KH * _L), jnp.float32),
            pltpu.VMEM((_CHUNK, _KH * _L), jnp.float32),
            pltpu.VMEM((b_per_w,), jnp.float32),
            pltpu.VMEM((b_per_w,), jnp.float32),
            pltpu.VMEM((_L,), jnp.float32),
            pltpu.VMEM((2 * _KH * _NTAIL,), jnp.float32),
            pltpu.VMEM((b_per_w,), jnp.float32),
            pltpu.SemaphoreType.DMA,
            pltpu.SemaphoreType.DMA,
            pltpu.SemaphoreType.DMA,
        ],
        compiler_params=pltpu.CompilerParams(
            needs_layout_passes=False, use_tc_tiling_on_sc=True),
    )
    emb0t = emb0.T.reshape(_NC, _KH, _V)
    emb1t = emb1.T.reshape(_NC, _KH, _V)
    bias16 = jnp.broadcast_to(bias, (_L,))
    tail = emb0[_ALIGNED_END:, :].T.reshape(-1)
    outs = run(user, item, fc.reshape(-1), bias16, tail, emb0t, emb1t)
    return outs[4] + outs[5]
